# baseline (speedup 1.0000x reference)
import functools

import jax
import jax.numpy as jnp
from jax import lax
from jax.experimental import pallas as pl
from jax.experimental.pallas import tpu as pltpu
from jax.experimental.pallas import tpu_sc as plsc

BATCH = 16384
EMBED = 32
NC = 2
NS = 16
NW = NC * NS
B_PER_W = BATCH // NW
CHUNK = 128
N_CHUNKS = B_PER_W // CHUNK


def _sc_gather_body(user_table, business_table, uid, bid, out_u, out_b,
                    idx_u, idx_b, rows_u, rows_b, sem):
    wid = lax.axis_index("c") * NS + lax.axis_index("s")
    base = wid * B_PER_W
    pltpu.sync_copy(uid.at[pl.ds(base, B_PER_W)], idx_u)
    pltpu.sync_copy(bid.at[pl.ds(base, B_PER_W)], idx_b)
    copies = []
    for c in range(N_CHUNKS):
        sl = pl.ds(c * CHUNK, CHUNK)
        copies.append(pltpu.async_copy(
            user_table.at[idx_u.at[sl]], rows_u.at[sl], sem))
        copies.append(pltpu.async_copy(
            business_table.at[idx_b.at[sl]], rows_b.at[sl], sem))
    for cp in copies:
        cp.wait()
    pltpu.sync_copy(rows_u, out_u.at[pl.ds(base, B_PER_W)])
    pltpu.sync_copy(rows_b, out_b.at[pl.ds(base, B_PER_W)])


@functools.cache
def _sc_gather():
    return pl.kernel(
        _sc_gather_body,
        out_type=[
            jax.ShapeDtypeStruct((BATCH, EMBED), jnp.float32),
            jax.ShapeDtypeStruct((BATCH, EMBED), jnp.float32),
        ],
        mesh=plsc.VectorSubcoreMesh(core_axis_name="c", subcore_axis_name="s"),
        compiler_params=pltpu.CompilerParams(use_tc_tiling_on_sc=False),
        scratch_types=[
            pltpu.VMEM((B_PER_W,), jnp.int32),
            pltpu.VMEM((B_PER_W,), jnp.int32),
            pltpu.VMEM((B_PER_W, EMBED), jnp.float32),
            pltpu.VMEM((B_PER_W, EMBED), jnp.float32),
            pltpu.SemaphoreType.DMA,
        ],
    )


def _mlp_body(u_ref, v_ref, w1u_ref, w1b_ref, b1_ref, w2_ref, b2_ref, o_ref):
    h = (jnp.dot(u_ref[...], w1u_ref[...], preferred_element_type=jnp.float32)
         + jnp.dot(v_ref[...], w1b_ref[...], preferred_element_type=jnp.float32)
         + b1_ref[...])
    h = jnp.maximum(h, 0.0)
    o = jnp.dot(h, w2_ref[...], preferred_element_type=jnp.float32) + b2_ref[...]
    o_ref[...] = jnp.maximum(o, 0.0)


def _mlp(uvec, bvec, w1u_t, w1b_t, b1, w2_t, b2, block=2048):
    n_blocks = BATCH // block
    return pl.pallas_call(
        _mlp_body,
        grid=(n_blocks,),
        in_specs=[
            pl.BlockSpec((block, EMBED), lambda i: (i, 0)),
            pl.BlockSpec((block, EMBED), lambda i: (i, 0)),
            pl.BlockSpec(w1u_t.shape, lambda i: (0, 0)),
            pl.BlockSpec(w1b_t.shape, lambda i: (0, 0)),
            pl.BlockSpec(b1.shape, lambda i: (0, 0)),
            pl.BlockSpec(w2_t.shape, lambda i: (0, 0)),
            pl.BlockSpec(b2.shape, lambda i: (0, 0)),
        ],
        out_specs=pl.BlockSpec((block, w2_t.shape[1]), lambda i: (i, 0)),
        out_shape=jax.ShapeDtypeStruct((BATCH, w2_t.shape[1]), jnp.float32),
    )(uvec, bvec, w1u_t, w1b_t, b1, w2_t, b2)


def kernel(user_ids, business_ids, user_table, business_table, W1, b1, W2, b2):
    uvec, bvec = _sc_gather()(user_table, business_table,
                            user_ids.astype(jnp.int32),
                            business_ids.astype(jnp.int32))
    w1u_t = W1[:, :EMBED].T
    w1b_t = W1[:, EMBED:].T
    w2_t = W2.T
    return _mlp(uvec, bvec, w1u_t, w1b_t, b1.reshape(1, -1), w2_t,
                b2.reshape(1, -1))
